# Initial kernel scaffold; baseline (speedup 1.0000x reference)
import functools

import jax
import jax.numpy as jnp
from jax import lax
from jax.experimental import pallas as pl
from jax.experimental.pallas import tpu as pltpu
from jax.experimental.pallas import tpu_sc as plsc

N = 10000
E = 320000
DF = 128
DO = 32
NC, NS = 2, 16
NW = NC * NS
NPAD = 10240
RPT = NPAD // NS
TE = 10240
EPAD = NW * TE
G = 128
NGRP = TE // G
NB = 2


def _sc_mesh():
    return plsc.VectorSubcoreMesh(
        core_axis_name="c", subcore_axis_name="s", num_cores=NC, num_subcores=NS
    )


@functools.partial(
    pl.kernel,
    out_type=jax.ShapeDtypeStruct((NC, NPAD), jnp.float32),
    mesh=_sc_mesh(),
    scratch_types=[
        pltpu.VMEM_SHARED((NPAD,), jnp.float32),
        pltpu.VMEM((G,), jnp.int32),
        pltpu.VMEM((G,), jnp.float32),
    ],
)
def _deg_kernel(dst_hbm, zero_hbm, out_hbm, acc, idx, ones):
    c = lax.axis_index("c")
    s = lax.axis_index("s")
    base = (c * NS + s) * TE
    r0 = s * RPT
    pltpu.sync_copy(zero_hbm.at[pl.ds(r0, RPT)], acc.at[pl.ds(r0, RPT)])
    for i in range(G // 16):
        ones[pl.ds(i * 16, 16)] = jnp.full((16,), 1.0, jnp.float32)
    plsc.subcore_barrier()

    def body(g, carry):
        off = base + g * G
        pltpu.sync_copy(dst_hbm.at[pl.ds(off, G)], idx)
        pltpu.sync_copy(ones, acc.at[idx], add=True)
        return carry

    lax.fori_loop(0, NGRP, body, 0)
    plsc.subcore_barrier()
    pltpu.sync_copy(acc.at[pl.ds(r0, RPT)], out_hbm.at[c].at[pl.ds(r0, RPT)])


def _make_agg(width):
    @functools.partial(
        pl.kernel,
        out_type=jax.ShapeDtypeStruct((NC, NPAD, width), jnp.float32),
        mesh=_sc_mesh(),
        scratch_types=[
            pltpu.VMEM_SHARED((NPAD, width), jnp.float32),
            [pltpu.VMEM((G,), jnp.int32) for _ in range(NB)],
            [pltpu.VMEM((G,), jnp.int32) for _ in range(NB)],
            [pltpu.VMEM((G, width), jnp.float32) for _ in range(NB)],
            [pltpu.SemaphoreType.DMA for _ in range(NB)],
        ],
    )
    def agg(src_hbm, dst_hbm, tbl_hbm, zero_hbm, out_hbm, acc, sidx, didx, rows, gsem):
        c = lax.axis_index("c")
        s = lax.axis_index("s")
        base = (c * NS + s) * TE
        r0 = s * RPT
        pltpu.sync_copy(zero_hbm.at[pl.ds(r0, RPT)], acc.at[pl.ds(r0, RPT)])
        plsc.subcore_barrier()

        def body(g2, carry):
            descs = []
            for b in range(NB):
                off = base + (g2 * NB + b) * G
                pltpu.sync_copy(src_hbm.at[pl.ds(off, G)], sidx[b])
                descs.append(pltpu.async_copy(tbl_hbm.at[sidx[b]], rows[b], gsem[b]))
            for b in range(NB):
                off = base + (g2 * NB + b) * G
                pltpu.sync_copy(dst_hbm.at[pl.ds(off, G)], didx[b])
                descs[b].wait()
                pltpu.sync_copy(rows[b], acc.at[didx[b]], add=True)
            return carry

        lax.fori_loop(0, NGRP // NB, body, 0)
        plsc.subcore_barrier()
        pltpu.sync_copy(acc.at[pl.ds(r0, RPT)], out_hbm.at[c].at[pl.ds(r0, RPT)])

    return agg


_agg128 = _make_agg(DF)
_agg32 = _make_agg(DO)


@functools.partial(
    pl.kernel,
    out_type=jax.ShapeDtypeStruct((EPAD,), jnp.float32),
    mesh=_sc_mesh(),
    scratch_types=[
        pltpu.VMEM((G,), jnp.int32),
        pltpu.VMEM((G,), jnp.int32),
        pltpu.VMEM((G, DO), jnp.float32),
        pltpu.VMEM((G, DO), jnp.float32),
        pltpu.VMEM((16, 16), jnp.float32),
        pltpu.VMEM((G,), jnp.float32),
        pltpu.SemaphoreType.DMA,
        pltpu.SemaphoreType.DMA,
    ],
)
def _decode_kernel(src_hbm, dst_hbm, zw_hbm, z_hbm, out_hbm,
                   sidx, didx, arows, brows, tmat, obuf, sema, semb):
    c = lax.axis_index("c")
    s = lax.axis_index("s")
    base = (c * NS + s) * TE
    lanes = lax.broadcasted_iota(jnp.int32, (16,), 0)

    def body(g, carry):
        off = base + g * G
        pltpu.sync_copy(src_hbm.at[pl.ds(off, G)], sidx)
        pltpu.sync_copy(dst_hbm.at[pl.ds(off, G)], didx)
        da = pltpu.async_copy(zw_hbm.at[sidx], arows, sema)
        db = pltpu.async_copy(z_hbm.at[didx], brows, semb)
        da.wait()
        db.wait()
        for e16 in range(G // 16):
            for l in range(16):
                e = e16 * 16 + l
                q = (arows[e, pl.ds(0, 16)] * brows[e, pl.ds(0, 16)]
                     + arows[e, pl.ds(16, 16)] * brows[e, pl.ds(16, 16)])
                plsc.store_scatter(tmat, [lanes, jnp.full((16,), l, jnp.int32)], q)
            ssum = tmat[0, pl.ds(0, 16)]
            for r in range(1, 16):
                ssum = ssum + tmat[r, pl.ds(0, 16)]
            obuf[pl.ds(e16 * 16, 16)] = 1.0 / (1.0 + jnp.exp(-ssum))
        pltpu.sync_copy(obuf, out_hbm.at[pl.ds(off, G)])
        return carry

    lax.fori_loop(0, NGRP, body, 0)


_BLK = 256


def _tc1_body(x_ref, wc_ref, da_ref, db_ref, xws_ref, rs_ref):
    deg = jnp.clip(da_ref[...] + db_ref[...], 1.0, None)
    r = lax.rsqrt(deg)
    xw = jnp.dot(x_ref[...], wc_ref[...], preferred_element_type=jnp.float32)
    xws_ref[...] = xw * r
    rs_ref[...] = r


_tc1 = pl.pallas_call(
    _tc1_body,
    grid=(NPAD // _BLK,),
    in_specs=[
        pl.BlockSpec((_BLK, DF), lambda i: (i, 0)),
        pl.BlockSpec((DF, DF), lambda i: (0, 0)),
        pl.BlockSpec((_BLK, 1), lambda i: (i, 0)),
        pl.BlockSpec((_BLK, 1), lambda i: (i, 0)),
    ],
    out_specs=[
        pl.BlockSpec((_BLK, DF), lambda i: (i, 0)),
        pl.BlockSpec((_BLK, 1), lambda i: (i, 0)),
    ],
    out_shape=[
        jax.ShapeDtypeStruct((NPAD, DF), jnp.float32),
        jax.ShapeDtypeStruct((NPAD, 1), jnp.float32),
    ],
)


def _tc2_body(aa_ref, ab_ref, rs_ref, wm_ref, ms_ref):
    r = rs_ref[...]
    h = jnp.maximum((aa_ref[...] + ab_ref[...]) * r, 0.0)
    ms_ref[...] = jnp.dot(h, wm_ref[...], preferred_element_type=jnp.float32) * r


_tc2 = pl.pallas_call(
    _tc2_body,
    grid=(NPAD // _BLK,),
    in_specs=[
        pl.BlockSpec((_BLK, DF), lambda i: (i, 0)),
        pl.BlockSpec((_BLK, DF), lambda i: (i, 0)),
        pl.BlockSpec((_BLK, 1), lambda i: (i, 0)),
        pl.BlockSpec((DF, DO), lambda i: (0, 0)),
    ],
    out_specs=pl.BlockSpec((_BLK, DO), lambda i: (i, 0)),
    out_shape=jax.ShapeDtypeStruct((NPAD, DO), jnp.float32),
)


def _tc3_body(za_ref, zb_ref, rs_ref, t_ref, z_ref, zw_ref):
    r = rs_ref[...]
    z = (za_ref[...] + zb_ref[...]) * r
    t = t_ref[0, 0]
    col = lax.broadcasted_iota(jnp.int32, (1, DO), 1)
    w = jnp.where(col < DO // 2, t, 1.0 - t)
    z_ref[...] = z
    zw_ref[...] = z * w


_tc3 = pl.pallas_call(
    _tc3_body,
    grid=(NPAD // _BLK,),
    in_specs=[
        pl.BlockSpec((_BLK, DO), lambda i: (i, 0)),
        pl.BlockSpec((_BLK, DO), lambda i: (i, 0)),
        pl.BlockSpec((_BLK, 1), lambda i: (i, 0)),
        pl.BlockSpec((1, 1), lambda i: (0, 0)),
    ],
    out_specs=[
        pl.BlockSpec((_BLK, DO), lambda i: (i, 0)),
        pl.BlockSpec((_BLK, DO), lambda i: (i, 0)),
    ],
    out_shape=[
        jax.ShapeDtypeStruct((NPAD, DO), jnp.float32),
        jax.ShapeDtypeStruct((NPAD, DO), jnp.float32),
    ],
)


def kernel(x, edge_index, temp, W1, W1mu, W1ls, W2, W2mu, W2ls):
    src = edge_index[0].astype(jnp.int32)
    dst = edge_index[1].astype(jnp.int32)
    padi = jnp.full((EPAD - E,), N, jnp.int32)
    srcp = jnp.concatenate([src, padi])
    dstp = jnp.concatenate([dst, padi])
    xp = jnp.zeros((NPAD, DF), jnp.float32).at[:N].set(x)
    Wc = jnp.concatenate([W1, W2], axis=1)
    Wm = (jnp.zeros((DF, DO), jnp.float32)
          .at[: DF // 2, : DO // 2].set(W1mu)
          .at[DF // 2 :, DO // 2 :].set(W2mu))
    zero_vec = jnp.zeros((NPAD,), jnp.float32)
    zero128 = jnp.zeros((NPAD, DF), jnp.float32)
    zero32 = jnp.zeros((NPAD, DO), jnp.float32)

    deg2 = _deg_kernel(dstp, zero_vec)
    xws, rs = _tc1(xp, Wc, deg2[0][:, None], deg2[1][:, None])
    agg2 = _agg128(srcp, dstp, xws, zero128)
    ms = _tc2(agg2[0], agg2[1], rs, Wm)
    aggz = _agg32(srcp, dstp, ms, zero32)
    z, zw = _tc3(aggz[0], aggz[1], rs, temp.reshape(1, 1))
    pred = _decode_kernel(srcp, dstp, zw, z)
    return pred[:E]


# speedup vs baseline: 10.3813x; 10.3813x over previous
import functools

import jax
import jax.numpy as jnp
from jax import lax
from jax.experimental import pallas as pl
from jax.experimental.pallas import tpu as pltpu
from jax.experimental.pallas import tpu_sc as plsc

N = 10000
E = 320000
DF = 128
DO = 32
NC, NS = 2, 16
NW = NC * NS
NPAD = 10240
RPT = NPAD // NS
TE = 10240
EPAD = NW * TE
G = 128
NGRP = TE // G
SB = 16
NSB = NGRP // SB


def _sc_mesh():
    return plsc.VectorSubcoreMesh(
        core_axis_name="c", subcore_axis_name="s", num_cores=NC, num_subcores=NS
    )


@functools.partial(
    pl.kernel,
    out_type=jax.ShapeDtypeStruct((NC, NPAD), jnp.float32),
    mesh=_sc_mesh(),
    scratch_types=[
        pltpu.VMEM_SHARED((NPAD,), jnp.float32),
        [pltpu.VMEM((G,), jnp.int32) for _ in range(4)],
        pltpu.VMEM((G,), jnp.float32),
        [pltpu.SemaphoreType.DMA for _ in range(4)],
    ],
)
def _deg_kernel(dst_hbm, zero_hbm, out_hbm, acc, dbuf, ones, sems):
    c = lax.axis_index("c")
    s = lax.axis_index("s")
    base = (c * NS + s) * TE
    r0 = s * RPT
    pltpu.sync_copy(zero_hbm.at[pl.ds(r0, RPT)], acc.at[pl.ds(r0, RPT)])
    for i in range(G // 16):
        ones[pl.ds(i * 16, 16)] = jnp.full((16,), 1.0, jnp.float32)
    plsc.subcore_barrier()

    def body(g4, carry):
        descs = []
        for b in range(4):
            off = base + (g4 * 4 + b) * G
            pltpu.sync_copy(dst_hbm.at[pl.ds(off, G)], dbuf[b])
            descs.append(
                pltpu.async_copy(ones, acc.at[dbuf[b]], sems[b], add=True)
            )
        for d in descs:
            d.wait()
        return carry

    lax.fori_loop(0, NGRP // 4, body, 0)
    plsc.subcore_barrier()
    pltpu.sync_copy(acc.at[pl.ds(r0, RPT)], out_hbm.at[c].at[pl.ds(r0, RPT)])


def _make_agg(width, nb):
    @functools.partial(
        pl.kernel,
        out_type=jax.ShapeDtypeStruct((NC, NPAD, width), jnp.float32),
        mesh=_sc_mesh(),
        scratch_types=[
            pltpu.VMEM_SHARED((NPAD, width), jnp.float32),
            pltpu.VMEM((SB, 1, G), jnp.int32),
            pltpu.VMEM((SB, 1, G), jnp.int32),
            [pltpu.VMEM((G, width), jnp.float32) for _ in range(nb)],
            [pltpu.SemaphoreType.DMA for _ in range(nb)],
            [pltpu.SemaphoreType.DMA for _ in range(nb)],
        ],
        compiler_params=pltpu.CompilerParams(use_tc_tiling_on_sc=False),
    )
    def agg(src_hbm, dst_hbm, tbl_hbm, zero_hbm, out_hbm, acc,
            sbuf, dbuf, rows, gsem, ssem):
        c = lax.axis_index("c")
        s = lax.axis_index("s")
        bgrp = (c * NS + s) * NGRP
        r0 = s * RPT
        pltpu.sync_copy(zero_hbm.at[pl.ds(r0, RPT)], acc.at[pl.ds(r0, RPT)])
        plsc.subcore_barrier()

        def body(sb, carry):
            g0 = bgrp + sb * SB
            pltpu.sync_copy(src_hbm.at[pl.ds(g0, SB)], sbuf)
            pltpu.sync_copy(dst_hbm.at[pl.ds(g0, SB)], dbuf)
            for j2 in range(SB // nb):
                gd, sd = [], []
                for b in range(nb):
                    j = j2 * nb + b
                    gd.append(
                        pltpu.async_copy(tbl_hbm.at[sbuf.at[j, 0]], rows[b], gsem[b])
                    )
                for b in range(nb):
                    j = j2 * nb + b
                    gd[b].wait()
                    sd.append(
                        pltpu.async_copy(rows[b], acc.at[dbuf.at[j, 0]], ssem[b],
                                         add=True)
                    )
                for b in range(nb):
                    sd[b].wait()
            return carry

        lax.fori_loop(0, NSB, body, 0)
        plsc.subcore_barrier()
        pltpu.sync_copy(acc.at[pl.ds(r0, RPT)], out_hbm.at[c].at[pl.ds(r0, RPT)])

    return agg


_agg128 = _make_agg(DF, 2)
_agg32 = _make_agg(DO, 4)


@functools.partial(
    pl.kernel,
    out_type=jax.ShapeDtypeStruct((EPAD,), jnp.float32),
    mesh=_sc_mesh(),
    scratch_types=[
        pltpu.VMEM((SB, 1, G), jnp.int32),
        pltpu.VMEM((SB, 1, G), jnp.int32),
        [pltpu.VMEM((G, DO), jnp.float32) for _ in range(2)],
        [pltpu.VMEM((G, DO), jnp.float32) for _ in range(2)],
        pltpu.VMEM((SB * G,), jnp.float32),
        [pltpu.SemaphoreType.DMA for _ in range(2)],
        [pltpu.SemaphoreType.DMA for _ in range(2)],
    ],
    compiler_params=pltpu.CompilerParams(
        use_tc_tiling_on_sc=False, needs_layout_passes=False
    ),
)
def _decode_kernel(src_hbm, dst_hbm, zw_hbm, z_hbm, out_hbm,
                   sbuf, dbuf, arows, brows, obuf, sema, semb):
    c = lax.axis_index("c")
    s = lax.axis_index("s")
    bgrp = (c * NS + s) * NGRP
    lanes = lax.broadcasted_iota(jnp.int32, (16,), 0)

    def dot_chunk(j, buf):
        for e16 in range(G // 16):
            rows16 = lanes + e16 * 16
            accs = [None] * 4
            for cidx in range(DO):
                cvec = jnp.full((16,), cidx, jnp.int32)
                p = (plsc.load_gather(arows[buf], [rows16, cvec])
                     * plsc.load_gather(brows[buf], [rows16, cvec]))
                k = cidx % 4
                accs[k] = p if accs[k] is None else accs[k] + p
            tot = (accs[0] + accs[1]) + (accs[2] + accs[3])
            obuf[pl.ds(j * G + e16 * 16, 16)] = 1.0 / (1.0 + jnp.exp(-tot))

    def wait_pair(b):
        pltpu.make_async_copy(zw_hbm.at[sbuf.at[0, 0]], arows[b], sema[b]).wait()
        pltpu.make_async_copy(z_hbm.at[dbuf.at[0, 0]], brows[b], semb[b]).wait()

    def fire_pair(j, b):
        pltpu.async_copy(zw_hbm.at[sbuf.at[j, 0]], arows[b], sema[b])
        pltpu.async_copy(z_hbm.at[dbuf.at[j, 0]], brows[b], semb[b])

    def body(sb, carry):
        g0 = bgrp + sb * SB
        pltpu.sync_copy(src_hbm.at[pl.ds(g0, SB)], sbuf)
        pltpu.sync_copy(dst_hbm.at[pl.ds(g0, SB)], dbuf)
        for b in range(2):
            fire_pair(b, b)

        def inner(j2, carry2):
            for b in range(2):
                j = j2 * 2 + b
                wait_pair(b)
                dot_chunk(j, b)
                fire_pair(j + 2, b)
            return carry2

        lax.fori_loop(0, SB // 2 - 1, inner, 0)
        for b in range(2):
            wait_pair(b)
            dot_chunk(SB - 2 + b, b)
        pltpu.sync_copy(obuf, out_hbm.at[pl.ds(g0 * G, SB * G)])
        return carry

    lax.fori_loop(0, NSB, body, 0)


_BLK = 256


def _tc1_body(x_ref, wc_ref, da_ref, db_ref, xws_ref, rs_ref):
    deg = jnp.clip(da_ref[...] + db_ref[...], 1.0, None)
    r = lax.rsqrt(deg)
    xw = jnp.dot(x_ref[...], wc_ref[...], preferred_element_type=jnp.float32)
    xws_ref[...] = xw * r
    rs_ref[...] = r


_tc1 = pl.pallas_call(
    _tc1_body,
    grid=(NPAD // _BLK,),
    in_specs=[
        pl.BlockSpec((_BLK, DF), lambda i: (i, 0)),
        pl.BlockSpec((DF, DF), lambda i: (0, 0)),
        pl.BlockSpec((_BLK, 1), lambda i: (i, 0)),
        pl.BlockSpec((_BLK, 1), lambda i: (i, 0)),
    ],
    out_specs=[
        pl.BlockSpec((_BLK, DF), lambda i: (i, 0)),
        pl.BlockSpec((_BLK, 1), lambda i: (i, 0)),
    ],
    out_shape=[
        jax.ShapeDtypeStruct((NPAD, DF), jnp.float32),
        jax.ShapeDtypeStruct((NPAD, 1), jnp.float32),
    ],
)


def _tc2_body(aa_ref, ab_ref, rs_ref, wm_ref, ms_ref):
    r = rs_ref[...]
    h = jnp.maximum((aa_ref[...] + ab_ref[...]) * r, 0.0)
    ms_ref[...] = jnp.dot(h, wm_ref[...], preferred_element_type=jnp.float32) * r


_tc2 = pl.pallas_call(
    _tc2_body,
    grid=(NPAD // _BLK,),
    in_specs=[
        pl.BlockSpec((_BLK, DF), lambda i: (i, 0)),
        pl.BlockSpec((_BLK, DF), lambda i: (i, 0)),
        pl.BlockSpec((_BLK, 1), lambda i: (i, 0)),
        pl.BlockSpec((DF, DO), lambda i: (0, 0)),
    ],
    out_specs=pl.BlockSpec((_BLK, DO), lambda i: (i, 0)),
    out_shape=jax.ShapeDtypeStruct((NPAD, DO), jnp.float32),
)


def _tc3_body(za_ref, zb_ref, rs_ref, t_ref, z_ref, zw_ref):
    r = rs_ref[...]
    z = (za_ref[...] + zb_ref[...]) * r
    t = t_ref[0, 0]
    col = lax.broadcasted_iota(jnp.int32, (1, DO), 1)
    w = jnp.where(col < DO // 2, t, 1.0 - t)
    z_ref[...] = z
    zw_ref[...] = z * w


_tc3 = pl.pallas_call(
    _tc3_body,
    grid=(NPAD // _BLK,),
    in_specs=[
        pl.BlockSpec((_BLK, DO), lambda i: (i, 0)),
        pl.BlockSpec((_BLK, DO), lambda i: (i, 0)),
        pl.BlockSpec((_BLK, 1), lambda i: (i, 0)),
        pl.BlockSpec((1, 1), lambda i: (0, 0)),
    ],
    out_specs=[
        pl.BlockSpec((_BLK, DO), lambda i: (i, 0)),
        pl.BlockSpec((_BLK, DO), lambda i: (i, 0)),
    ],
    out_shape=[
        jax.ShapeDtypeStruct((NPAD, DO), jnp.float32),
        jax.ShapeDtypeStruct((NPAD, DO), jnp.float32),
    ],
)


def kernel(x, edge_index, temp, W1, W1mu, W1ls, W2, W2mu, W2ls):
    src = edge_index[0].astype(jnp.int32)
    dst = edge_index[1].astype(jnp.int32)
    padi = jnp.full((EPAD - E,), N, jnp.int32)
    srcf = jnp.concatenate([src, padi])
    dstf = jnp.concatenate([dst, padi])
    srcp = srcf.reshape(EPAD // G, 1, G)
    dstp = dstf.reshape(EPAD // G, 1, G)
    xp = jnp.zeros((NPAD, DF), jnp.float32).at[:N].set(x)
    Wc = jnp.concatenate([W1, W2], axis=1)
    Wm = (jnp.zeros((DF, DO), jnp.float32)
          .at[: DF // 2, : DO // 2].set(W1mu)
          .at[DF // 2 :, DO // 2 :].set(W2mu))
    zero_vec = jnp.zeros((NPAD,), jnp.float32)
    zero128 = jnp.zeros((NPAD, DF), jnp.float32)
    zero32 = jnp.zeros((NPAD, DO), jnp.float32)

    deg2 = _deg_kernel(dstf, zero_vec)
    xws, rs = _tc1(xp, Wc, deg2[0][:, None], deg2[1][:, None])
    agg2 = _agg128(srcp, dstp, xws, zero128)
    ms = _tc2(agg2[0], agg2[1], rs, Wm)
    aggz = _agg32(srcp, dstp, ms, zero32)
    z, zw = _tc3(aggz[0], aggz[1], rs, temp.reshape(1, 1))
    pred = _decode_kernel(srcp, dstp, zw, z)
    return pred[:E]


# speedup vs baseline: 12.4446x; 1.1987x over previous
import functools

import jax
import jax.numpy as jnp
from jax import lax
from jax.experimental import pallas as pl
from jax.experimental.pallas import tpu as pltpu
from jax.experimental.pallas import tpu_sc as plsc

N = 10000
E = 320000
DF = 128
DO = 32
NC, NS = 2, 16
NW = NC * NS
NPAD = 10240
RPT = NPAD // NS
TE = 10240
EPAD = NW * TE
G = 128
NGRP = TE // G
SB = 16
NSB = NGRP // SB


def _sc_mesh():
    return plsc.VectorSubcoreMesh(
        core_axis_name="c", subcore_axis_name="s", num_cores=NC, num_subcores=NS
    )


@functools.partial(
    pl.kernel,
    out_type=jax.ShapeDtypeStruct((NC, NPAD), jnp.float32),
    mesh=_sc_mesh(),
    scratch_types=[
        pltpu.VMEM_SHARED((NPAD,), jnp.float32),
        [pltpu.VMEM((G,), jnp.int32) for _ in range(4)],
        pltpu.VMEM((G,), jnp.float32),
        [pltpu.SemaphoreType.DMA for _ in range(4)],
    ],
)
def _deg_kernel(dst_hbm, zero_hbm, out_hbm, acc, dbuf, ones, sems):
    c = lax.axis_index("c")
    s = lax.axis_index("s")
    base = (c * NS + s) * TE
    r0 = s * RPT
    pltpu.sync_copy(zero_hbm.at[pl.ds(r0, RPT)], acc.at[pl.ds(r0, RPT)])
    for i in range(G // 16):
        ones[pl.ds(i * 16, 16)] = jnp.full((16,), 1.0, jnp.float32)
    plsc.subcore_barrier()

    def body(g4, carry):
        descs = []
        for b in range(4):
            off = base + (g4 * 4 + b) * G
            pltpu.sync_copy(dst_hbm.at[pl.ds(off, G)], dbuf[b])
            descs.append(
                pltpu.async_copy(ones, acc.at[dbuf[b]], sems[b], add=True)
            )
        for d in descs:
            d.wait()
        return carry

    lax.fori_loop(0, NGRP // 4, body, 0)
    plsc.subcore_barrier()
    pltpu.sync_copy(acc.at[pl.ds(r0, RPT)], out_hbm.at[c].at[pl.ds(r0, RPT)])


def _make_agg(width, nb, dtype, spmem_table):
    scratch = [
        pltpu.VMEM_SHARED((NPAD, width), dtype),
        pltpu.VMEM((SB, 1, G), jnp.int32),
        pltpu.VMEM((SB, 1, G), jnp.int32),
        [pltpu.VMEM((G, width), dtype) for _ in range(nb)],
        [pltpu.SemaphoreType.DMA for _ in range(nb)],
        [pltpu.SemaphoreType.DMA for _ in range(nb)],
    ]
    if spmem_table:
        scratch.append(pltpu.VMEM_SHARED((NPAD, width), dtype))

    @functools.partial(
        pl.kernel,
        out_type=jax.ShapeDtypeStruct((NC, NPAD, width), dtype),
        mesh=_sc_mesh(),
        scratch_types=scratch,
        compiler_params=pltpu.CompilerParams(use_tc_tiling_on_sc=False),
    )
    def agg(src_hbm, dst_hbm, tbl_hbm, zero_hbm, out_hbm, acc,
            sbuf, dbuf, rows, gsem, ssem, *maybe_tbl):
        c = lax.axis_index("c")
        s = lax.axis_index("s")
        bgrp = (c * NS + s) * NGRP
        r0 = s * RPT
        pltpu.sync_copy(zero_hbm.at[pl.ds(r0, RPT)], acc.at[pl.ds(r0, RPT)])
        if spmem_table:
            tbl = maybe_tbl[0]
            pltpu.sync_copy(tbl_hbm.at[pl.ds(r0, RPT)], tbl.at[pl.ds(r0, RPT)])
        else:
            tbl = tbl_hbm
        plsc.subcore_barrier()

        def body(sb, carry):
            g0 = bgrp + sb * SB
            pltpu.sync_copy(src_hbm.at[pl.ds(g0, SB)], sbuf)
            pltpu.sync_copy(dst_hbm.at[pl.ds(g0, SB)], dbuf)
            for j2 in range(SB // nb):
                gd, sd = [], []
                for b in range(nb):
                    j = j2 * nb + b
                    gd.append(
                        pltpu.async_copy(tbl.at[sbuf.at[j, 0]], rows[b], gsem[b])
                    )
                for b in range(nb):
                    j = j2 * nb + b
                    gd[b].wait()
                    sd.append(
                        pltpu.async_copy(rows[b], acc.at[dbuf.at[j, 0]], ssem[b],
                                         add=True)
                    )
                for b in range(nb):
                    sd[b].wait()
            return carry

        lax.fori_loop(0, NSB, body, 0)
        plsc.subcore_barrier()
        pltpu.sync_copy(acc.at[pl.ds(r0, RPT)], out_hbm.at[c].at[pl.ds(r0, RPT)])

    return agg


_agg128 = _make_agg(DF, 4, jnp.bfloat16, False)
_agg32 = _make_agg(DO, 4, jnp.float32, True)


@functools.partial(
    pl.kernel,
    out_type=jax.ShapeDtypeStruct((EPAD,), jnp.float32),
    mesh=_sc_mesh(),
    scratch_types=[
        pltpu.VMEM((SB, 1, G), jnp.int32),
        pltpu.VMEM((SB, 1, G), jnp.int32),
        [pltpu.VMEM((G, DO), jnp.float32) for _ in range(2)],
        [pltpu.VMEM((G, DO), jnp.float32) for _ in range(2)],
        pltpu.VMEM((SB * G,), jnp.float32),
        [pltpu.SemaphoreType.DMA for _ in range(2)],
        [pltpu.SemaphoreType.DMA for _ in range(2)],
        pltpu.VMEM_SHARED((NPAD, DO), jnp.float32),
        pltpu.VMEM_SHARED((NPAD, DO), jnp.float32),
    ],
    compiler_params=pltpu.CompilerParams(
        use_tc_tiling_on_sc=False, needs_layout_passes=False
    ),
)
def _decode_kernel(src_hbm, dst_hbm, zw_hbm, z_hbm, out_hbm,
                   sbuf, dbuf, arows, brows, obuf, sema, semb, zws, zs):
    c = lax.axis_index("c")
    s = lax.axis_index("s")
    bgrp = (c * NS + s) * NGRP
    r0 = s * RPT
    pltpu.sync_copy(zw_hbm.at[pl.ds(r0, RPT)], zws.at[pl.ds(r0, RPT)])
    pltpu.sync_copy(z_hbm.at[pl.ds(r0, RPT)], zs.at[pl.ds(r0, RPT)])
    plsc.subcore_barrier()
    lanes = lax.broadcasted_iota(jnp.int32, (16,), 0)

    def dot_chunk(j, buf):
        for e16 in range(G // 16):
            rows16 = lanes + e16 * 16
            accs = [None] * 4
            for cidx in range(DO):
                cvec = jnp.full((16,), cidx, jnp.int32)
                p = (plsc.load_gather(arows[buf], [rows16, cvec])
                     * plsc.load_gather(brows[buf], [rows16, cvec]))
                k = cidx % 4
                accs[k] = p if accs[k] is None else accs[k] + p
            tot = (accs[0] + accs[1]) + (accs[2] + accs[3])
            obuf[pl.ds(j * G + e16 * 16, 16)] = 1.0 / (1.0 + jnp.exp(-tot))

    def wait_pair(b):
        pltpu.make_async_copy(zws.at[sbuf.at[0, 0]], arows[b], sema[b]).wait()
        pltpu.make_async_copy(zs.at[dbuf.at[0, 0]], brows[b], semb[b]).wait()

    def fire_pair(j, b):
        pltpu.async_copy(zws.at[sbuf.at[j, 0]], arows[b], sema[b])
        pltpu.async_copy(zs.at[dbuf.at[j, 0]], brows[b], semb[b])

    def body(sb, carry):
        g0 = bgrp + sb * SB
        pltpu.sync_copy(src_hbm.at[pl.ds(g0, SB)], sbuf)
        pltpu.sync_copy(dst_hbm.at[pl.ds(g0, SB)], dbuf)
        for b in range(2):
            fire_pair(b, b)

        def inner(j2, carry2):
            for b in range(2):
                j = j2 * 2 + b
                wait_pair(b)
                dot_chunk(j, b)
                fire_pair(j + 2, b)
            return carry2

        lax.fori_loop(0, SB // 2 - 1, inner, 0)
        for b in range(2):
            wait_pair(b)
            dot_chunk(SB - 2 + b, b)
        pltpu.sync_copy(obuf, out_hbm.at[pl.ds(g0 * G, SB * G)])
        return carry

    lax.fori_loop(0, NSB, body, 0)


_BLK = 256


def _tc1_body(x_ref, wc_ref, da_ref, db_ref, xws_ref, rs_ref):
    deg = jnp.clip(da_ref[...] + db_ref[...], 1.0, None)
    r = lax.rsqrt(deg)
    xw = jnp.dot(x_ref[...], wc_ref[...], preferred_element_type=jnp.float32)
    xws_ref[...] = (xw * r).astype(jnp.bfloat16)
    rs_ref[...] = r


_tc1 = pl.pallas_call(
    _tc1_body,
    grid=(NPAD // _BLK,),
    in_specs=[
        pl.BlockSpec((_BLK, DF), lambda i: (i, 0)),
        pl.BlockSpec((DF, DF), lambda i: (0, 0)),
        pl.BlockSpec((_BLK, 1), lambda i: (i, 0)),
        pl.BlockSpec((_BLK, 1), lambda i: (i, 0)),
    ],
    out_specs=[
        pl.BlockSpec((_BLK, DF), lambda i: (i, 0)),
        pl.BlockSpec((_BLK, 1), lambda i: (i, 0)),
    ],
    out_shape=[
        jax.ShapeDtypeStruct((NPAD, DF), jnp.bfloat16),
        jax.ShapeDtypeStruct((NPAD, 1), jnp.float32),
    ],
)


def _tc2_body(aa_ref, ab_ref, rs_ref, wm_ref, ms_ref):
    r = rs_ref[...]
    agg = aa_ref[...].astype(jnp.float32) + ab_ref[...].astype(jnp.float32)
    h = jnp.maximum(agg * r, 0.0)
    ms_ref[...] = jnp.dot(h, wm_ref[...], preferred_element_type=jnp.float32) * r


_tc2 = pl.pallas_call(
    _tc2_body,
    grid=(NPAD // _BLK,),
    in_specs=[
        pl.BlockSpec((_BLK, DF), lambda i: (i, 0)),
        pl.BlockSpec((_BLK, DF), lambda i: (i, 0)),
        pl.BlockSpec((_BLK, 1), lambda i: (i, 0)),
        pl.BlockSpec((DF, DO), lambda i: (0, 0)),
    ],
    out_specs=pl.BlockSpec((_BLK, DO), lambda i: (i, 0)),
    out_shape=jax.ShapeDtypeStruct((NPAD, DO), jnp.float32),
)


def _tc3_body(za_ref, zb_ref, rs_ref, t_ref, z_ref, zw_ref):
    r = rs_ref[...]
    z = (za_ref[...] + zb_ref[...]) * r
    t = t_ref[0, 0]
    col = lax.broadcasted_iota(jnp.int32, (1, DO), 1)
    w = jnp.where(col < DO // 2, t, 1.0 - t)
    z_ref[...] = z
    zw_ref[...] = z * w


_tc3 = pl.pallas_call(
    _tc3_body,
    grid=(NPAD // _BLK,),
    in_specs=[
        pl.BlockSpec((_BLK, DO), lambda i: (i, 0)),
        pl.BlockSpec((_BLK, DO), lambda i: (i, 0)),
        pl.BlockSpec((_BLK, 1), lambda i: (i, 0)),
        pl.BlockSpec((1, 1), lambda i: (0, 0)),
    ],
    out_specs=[
        pl.BlockSpec((_BLK, DO), lambda i: (i, 0)),
        pl.BlockSpec((_BLK, DO), lambda i: (i, 0)),
    ],
    out_shape=[
        jax.ShapeDtypeStruct((NPAD, DO), jnp.float32),
        jax.ShapeDtypeStruct((NPAD, DO), jnp.float32),
    ],
)


def kernel(x, edge_index, temp, W1, W1mu, W1ls, W2, W2mu, W2ls):
    src = edge_index[0].astype(jnp.int32)
    dst = edge_index[1].astype(jnp.int32)
    padi = jnp.full((EPAD - E,), N, jnp.int32)
    srcf = jnp.concatenate([src, padi])
    dstf = jnp.concatenate([dst, padi])
    srcp = srcf.reshape(EPAD // G, 1, G)
    dstp = dstf.reshape(EPAD // G, 1, G)
    xp = jnp.zeros((NPAD, DF), jnp.float32).at[:N].set(x)
    Wc = jnp.concatenate([W1, W2], axis=1)
    Wm = (jnp.zeros((DF, DO), jnp.float32)
          .at[: DF // 2, : DO // 2].set(W1mu)
          .at[DF // 2 :, DO // 2 :].set(W2mu))
    zero_vec = jnp.zeros((NPAD,), jnp.float32)
    zero128 = jnp.zeros((NPAD, DF), jnp.bfloat16)
    zero32 = jnp.zeros((NPAD, DO), jnp.float32)

    deg2 = _deg_kernel(dstf, zero_vec)
    xws, rs = _tc1(xp, Wc, deg2[0][:, None], deg2[1][:, None])
    agg2 = _agg128(srcp, dstp, xws, zero128)
    ms = _tc2(agg2[0], agg2[1], rs, Wm)
    aggz = _agg32(srcp, dstp, ms, zero32)
    z, zw = _tc3(aggz[0], aggz[1], rs, temp.reshape(1, 1))
    pred = _decode_kernel(srcp, dstp, zw, z)
    return pred[:E]


# speedup vs baseline: 21.3430x; 1.7150x over previous
import functools

import jax
import jax.numpy as jnp
from jax import lax
from jax.experimental import pallas as pl
from jax.experimental.pallas import tpu as pltpu
from jax.experimental.pallas import tpu_sc as plsc

N = 10000
E = 320000
DF = 128
DO = 32
NC, NS = 2, 16
NW = NC * NS
NPAD = 10240
RPT = NPAD // NS
TE = 10240
EPAD = NW * TE
G = 128
NGRP = TE // G
SB = 16
NSB = NGRP // SB


def _sc_mesh():
    return plsc.VectorSubcoreMesh(
        core_axis_name="c", subcore_axis_name="s", num_cores=NC, num_subcores=NS
    )


@functools.partial(
    pl.kernel,
    out_type=jax.ShapeDtypeStruct((NC, NPAD), jnp.float32),
    mesh=_sc_mesh(),
    scratch_types=[
        pltpu.VMEM_SHARED((NPAD,), jnp.float32),
        [pltpu.VMEM((G,), jnp.int32) for _ in range(4)],
        pltpu.VMEM((G,), jnp.float32),
        [pltpu.SemaphoreType.DMA for _ in range(4)],
    ],
)
def _deg_kernel(dst_hbm, zero_hbm, out_hbm, acc, dbuf, ones, sems):
    c = lax.axis_index("c")
    s = lax.axis_index("s")
    base = (c * NS + s) * TE
    r0 = s * RPT
    pltpu.sync_copy(zero_hbm.at[pl.ds(r0, RPT)], acc.at[pl.ds(r0, RPT)])
    for i in range(G // 16):
        ones[pl.ds(i * 16, 16)] = jnp.full((16,), 1.0, jnp.float32)
    plsc.subcore_barrier()

    def body(g4, carry):
        descs = []
        for b in range(4):
            off = base + (g4 * 4 + b) * G
            pltpu.sync_copy(dst_hbm.at[pl.ds(off, G)], dbuf[b])
            descs.append(
                pltpu.async_copy(ones, acc.at[dbuf[b]], sems[b], add=True)
            )
        for d in descs:
            d.wait()
        return carry

    lax.fori_loop(0, NGRP // 4, body, 0)
    plsc.subcore_barrier()
    pltpu.sync_copy(acc.at[pl.ds(r0, RPT)], out_hbm.at[c].at[pl.ds(r0, RPT)])


def _make_agg(width, nb, dtype, spmem_table):
    scratch = [
        pltpu.VMEM_SHARED((NPAD, width), dtype),
        pltpu.VMEM((SB, 1, G), jnp.int32),
        pltpu.VMEM((SB, 1, G), jnp.int32),
        [pltpu.VMEM((G, width), dtype) for _ in range(nb)],
        [pltpu.SemaphoreType.DMA for _ in range(nb)],
        [pltpu.SemaphoreType.DMA for _ in range(nb)],
    ]
    if spmem_table:
        scratch.append(pltpu.VMEM_SHARED((NPAD, width), dtype))

    @functools.partial(
        pl.kernel,
        out_type=jax.ShapeDtypeStruct((NC, NPAD, width), dtype),
        mesh=_sc_mesh(),
        scratch_types=scratch,
        compiler_params=pltpu.CompilerParams(use_tc_tiling_on_sc=False),
    )
    def agg(src_hbm, dst_hbm, tbl_hbm, zero_hbm, out_hbm, acc,
            sbuf, dbuf, rows, gsem, ssem, *maybe_tbl):
        c = lax.axis_index("c")
        s = lax.axis_index("s")
        bgrp = (c * NS + s) * NGRP
        r0 = s * RPT
        pltpu.sync_copy(zero_hbm.at[pl.ds(r0, RPT)], acc.at[pl.ds(r0, RPT)])
        if spmem_table:
            tbl = maybe_tbl[0]
            pltpu.sync_copy(tbl_hbm.at[pl.ds(r0, RPT)], tbl.at[pl.ds(r0, RPT)])
        else:
            tbl = tbl_hbm
        plsc.subcore_barrier()

        def body(sb, carry):
            g0 = bgrp + sb * SB
            pltpu.sync_copy(src_hbm.at[pl.ds(g0, SB)], sbuf)
            pltpu.sync_copy(dst_hbm.at[pl.ds(g0, SB)], dbuf)
            for j2 in range(SB // nb):
                gd, sd = [], []
                for b in range(nb):
                    j = j2 * nb + b
                    gd.append(
                        pltpu.async_copy(tbl.at[sbuf.at[j, 0]], rows[b], gsem[b])
                    )
                for b in range(nb):
                    j = j2 * nb + b
                    gd[b].wait()
                    sd.append(
                        pltpu.async_copy(rows[b], acc.at[dbuf.at[j, 0]], ssem[b],
                                         add=True)
                    )
                for b in range(nb):
                    sd[b].wait()
            return carry

        lax.fori_loop(0, NSB, body, 0)
        plsc.subcore_barrier()
        pltpu.sync_copy(acc.at[pl.ds(r0, RPT)], out_hbm.at[c].at[pl.ds(r0, RPT)])

    return agg


_agg128 = _make_agg(DF, 2, jnp.bfloat16, True)
_agg32 = _make_agg(DO, 4, jnp.float32, True)


@functools.partial(
    pl.kernel,
    out_type=jax.ShapeDtypeStruct((EPAD,), jnp.float32),
    mesh=_sc_mesh(),
    scratch_types=[
        pltpu.VMEM((SB, 1, G), jnp.int32),
        pltpu.VMEM((SB, 1, G), jnp.int32),
        [pltpu.VMEM((G, DO), jnp.float32) for _ in range(2)],
        [pltpu.VMEM((G, DO), jnp.float32) for _ in range(2)],
        pltpu.VMEM((SB * G,), jnp.float32),
        [pltpu.SemaphoreType.DMA for _ in range(2)],
        [pltpu.SemaphoreType.DMA for _ in range(2)],
        pltpu.VMEM_SHARED((NPAD, DO), jnp.float32),
        pltpu.VMEM_SHARED((NPAD, DO), jnp.float32),
    ],
    compiler_params=pltpu.CompilerParams(
        use_tc_tiling_on_sc=False, needs_layout_passes=False
    ),
)
def _decode_kernel(src_hbm, dst_hbm, zw_hbm, z_hbm, out_hbm,
                   sbuf, dbuf, arows, brows, obuf, sema, semb, zws, zs):
    c = lax.axis_index("c")
    s = lax.axis_index("s")
    bgrp = (c * NS + s) * NGRP
    r0 = s * RPT
    pltpu.sync_copy(zw_hbm.at[pl.ds(r0, RPT)], zws.at[pl.ds(r0, RPT)])
    pltpu.sync_copy(z_hbm.at[pl.ds(r0, RPT)], zs.at[pl.ds(r0, RPT)])
    plsc.subcore_barrier()
    lanes = lax.broadcasted_iota(jnp.int32, (16,), 0)

    def dot_chunk(j, buf):
        for e16 in range(G // 16):
            rows16 = lanes + e16 * 16
            accs = [None] * 4
            for cidx in range(DO):
                cvec = jnp.bitwise_and(lanes + cidx, DO - 1)
                p = (plsc.load_gather(arows[buf], [rows16, cvec])
                     * plsc.load_gather(brows[buf], [rows16, cvec]))
                k = cidx % 4
                accs[k] = p if accs[k] is None else accs[k] + p
            tot = (accs[0] + accs[1]) + (accs[2] + accs[3])
            obuf[pl.ds(j * G + e16 * 16, 16)] = 1.0 / (1.0 + jnp.exp(-tot))

    def wait_pair(b):
        pltpu.make_async_copy(zws.at[sbuf.at[0, 0]], arows[b], sema[b]).wait()
        pltpu.make_async_copy(zs.at[dbuf.at[0, 0]], brows[b], semb[b]).wait()

    def fire_pair(j, b):
        pltpu.async_copy(zws.at[sbuf.at[j, 0]], arows[b], sema[b])
        pltpu.async_copy(zs.at[dbuf.at[j, 0]], brows[b], semb[b])

    def body(sb, carry):
        g0 = bgrp + sb * SB
        pltpu.sync_copy(src_hbm.at[pl.ds(g0, SB)], sbuf)
        pltpu.sync_copy(dst_hbm.at[pl.ds(g0, SB)], dbuf)
        for b in range(2):
            fire_pair(b, b)

        def inner(j2, carry2):
            for b in range(2):
                j = j2 * 2 + b
                wait_pair(b)
                dot_chunk(j, b)
                fire_pair(j + 2, b)
            return carry2

        lax.fori_loop(0, SB // 2 - 1, inner, 0)
        for b in range(2):
            wait_pair(b)
            dot_chunk(SB - 2 + b, b)
        pltpu.sync_copy(obuf, out_hbm.at[pl.ds(g0 * G, SB * G)])
        return carry

    lax.fori_loop(0, NSB, body, 0)


_BLK = 256


def _tc1_body(x_ref, wc_ref, da_ref, db_ref, xws_ref, rs_ref):
    deg = jnp.clip(da_ref[...] + db_ref[...], 1.0, None)
    r = lax.rsqrt(deg)
    xw = jnp.dot(x_ref[...], wc_ref[...], preferred_element_type=jnp.float32)
    xws_ref[...] = (xw * r).astype(jnp.bfloat16)
    rs_ref[...] = r


_tc1 = pl.pallas_call(
    _tc1_body,
    grid=(NPAD // _BLK,),
    in_specs=[
        pl.BlockSpec((_BLK, DF), lambda i: (i, 0)),
        pl.BlockSpec((DF, DF), lambda i: (0, 0)),
        pl.BlockSpec((_BLK, 1), lambda i: (i, 0)),
        pl.BlockSpec((_BLK, 1), lambda i: (i, 0)),
    ],
    out_specs=[
        pl.BlockSpec((_BLK, DF), lambda i: (i, 0)),
        pl.BlockSpec((_BLK, 1), lambda i: (i, 0)),
    ],
    out_shape=[
        jax.ShapeDtypeStruct((NPAD, DF), jnp.bfloat16),
        jax.ShapeDtypeStruct((NPAD, 1), jnp.float32),
    ],
)


def _tc2_body(aa_ref, ab_ref, rs_ref, wm_ref, ms_ref):
    r = rs_ref[...]
    agg = aa_ref[...].astype(jnp.float32) + ab_ref[...].astype(jnp.float32)
    h = jnp.maximum(agg * r, 0.0)
    ms_ref[...] = jnp.dot(h, wm_ref[...], preferred_element_type=jnp.float32) * r


_tc2 = pl.pallas_call(
    _tc2_body,
    grid=(NPAD // _BLK,),
    in_specs=[
        pl.BlockSpec((_BLK, DF), lambda i: (i, 0)),
        pl.BlockSpec((_BLK, DF), lambda i: (i, 0)),
        pl.BlockSpec((_BLK, 1), lambda i: (i, 0)),
        pl.BlockSpec((DF, DO), lambda i: (0, 0)),
    ],
    out_specs=pl.BlockSpec((_BLK, DO), lambda i: (i, 0)),
    out_shape=jax.ShapeDtypeStruct((NPAD, DO), jnp.float32),
)


def _tc3_body(za_ref, zb_ref, rs_ref, t_ref, z_ref, zw_ref):
    r = rs_ref[...]
    z = (za_ref[...] + zb_ref[...]) * r
    t = t_ref[0, 0]
    col = lax.broadcasted_iota(jnp.int32, (1, DO), 1)
    w = jnp.where(col < DO // 2, t, 1.0 - t)
    z_ref[...] = z
    zw_ref[...] = z * w


_tc3 = pl.pallas_call(
    _tc3_body,
    grid=(NPAD // _BLK,),
    in_specs=[
        pl.BlockSpec((_BLK, DO), lambda i: (i, 0)),
        pl.BlockSpec((_BLK, DO), lambda i: (i, 0)),
        pl.BlockSpec((_BLK, 1), lambda i: (i, 0)),
        pl.BlockSpec((1, 1), lambda i: (0, 0)),
    ],
    out_specs=[
        pl.BlockSpec((_BLK, DO), lambda i: (i, 0)),
        pl.BlockSpec((_BLK, DO), lambda i: (i, 0)),
    ],
    out_shape=[
        jax.ShapeDtypeStruct((NPAD, DO), jnp.float32),
        jax.ShapeDtypeStruct((NPAD, DO), jnp.float32),
    ],
)


def kernel(x, edge_index, temp, W1, W1mu, W1ls, W2, W2mu, W2ls):
    src = edge_index[0].astype(jnp.int32)
    dst = edge_index[1].astype(jnp.int32)
    padi = jnp.full((EPAD - E,), N, jnp.int32)
    srcf = jnp.concatenate([src, padi])
    dstf = jnp.concatenate([dst, padi])
    srcp = srcf.reshape(EPAD // G, 1, G)
    dstp = dstf.reshape(EPAD // G, 1, G)
    xp = jnp.zeros((NPAD, DF), jnp.float32).at[:N].set(x)
    Wc = jnp.concatenate([W1, W2], axis=1)
    Wm = (jnp.zeros((DF, DO), jnp.float32)
          .at[: DF // 2, : DO // 2].set(W1mu)
          .at[DF // 2 :, DO // 2 :].set(W2mu))
    zero_vec = jnp.zeros((NPAD,), jnp.float32)
    zero128 = jnp.zeros((NPAD, DF), jnp.bfloat16)
    zero32 = jnp.zeros((NPAD, DO), jnp.float32)

    deg2 = _deg_kernel(dstf, zero_vec)
    xws, rs = _tc1(xp, Wc, deg2[0][:, None], deg2[1][:, None])
    agg2 = _agg128(srcp, dstp, xws, zero128)
    ms = _tc2(agg2[0], agg2[1], rs, Wm)
    aggz = _agg32(srcp, dstp, ms, zero32)
    z, zw = _tc3(aggz[0], aggz[1], rs, temp.reshape(1, 1))
    pred = _decode_kernel(srcp, dstp, zw, z)
    return pred[:E]


# speedup vs baseline: 24.0481x; 1.1267x over previous
import functools

import jax
import jax.numpy as jnp
from jax import lax
from jax.experimental import pallas as pl
from jax.experimental.pallas import tpu as pltpu
from jax.experimental.pallas import tpu_sc as plsc

N = 10000
E = 320000
DF = 128
DO = 32
NC, NS = 2, 16
NW = NC * NS
NPAD = 10240
RPT = NPAD // NS
TE = 10240
EPAD = NW * TE
G = 128
NGRP = TE // G
SB = 16
NSB = NGRP // SB


def _sc_mesh():
    return plsc.VectorSubcoreMesh(
        core_axis_name="c", subcore_axis_name="s", num_cores=NC, num_subcores=NS
    )


@functools.partial(
    pl.kernel,
    out_type=jax.ShapeDtypeStruct((NC, NPAD), jnp.float32),
    mesh=_sc_mesh(),
    scratch_types=[
        pltpu.VMEM_SHARED((NPAD,), jnp.float32),
        [pltpu.VMEM((G,), jnp.int32) for _ in range(4)],
        pltpu.VMEM((G,), jnp.float32),
        [pltpu.SemaphoreType.DMA for _ in range(4)],
    ],
)
def _deg_kernel(dst_hbm, zero_hbm, out_hbm, acc, dbuf, ones, sems):
    c = lax.axis_index("c")
    s = lax.axis_index("s")
    base = (c * NS + s) * TE
    r0 = s * RPT
    pltpu.sync_copy(zero_hbm.at[pl.ds(r0, RPT)], acc.at[pl.ds(r0, RPT)])
    for i in range(G // 16):
        ones[pl.ds(i * 16, 16)] = jnp.full((16,), 1.0, jnp.float32)
    plsc.subcore_barrier()

    def body(g4, carry):
        descs = []
        for b in range(4):
            off = base + (g4 * 4 + b) * G
            pltpu.sync_copy(dst_hbm.at[pl.ds(off, G)], dbuf[b])
            descs.append(
                pltpu.async_copy(ones, acc.at[dbuf[b]], sems[b], add=True)
            )
        for d in descs:
            d.wait()
        return carry

    lax.fori_loop(0, NGRP // 4, body, 0)
    plsc.subcore_barrier()
    pltpu.sync_copy(acc.at[pl.ds(r0, RPT)], out_hbm.at[c].at[pl.ds(r0, RPT)])


def _make_agg(width, nb, dtype, spmem_table):
    scratch = [
        pltpu.VMEM_SHARED((NPAD, width), dtype),
        pltpu.VMEM((SB, 1, G), jnp.int32),
        pltpu.VMEM((SB, 1, G), jnp.int32),
        [pltpu.VMEM((G, width), dtype) for _ in range(nb)],
        [pltpu.SemaphoreType.DMA for _ in range(nb)],
        [pltpu.SemaphoreType.DMA for _ in range(nb)],
    ]
    if spmem_table:
        scratch.append(pltpu.VMEM_SHARED((NPAD, width), dtype))

    @functools.partial(
        pl.kernel,
        out_type=jax.ShapeDtypeStruct((NC, NPAD, width), dtype),
        mesh=_sc_mesh(),
        scratch_types=scratch,
        compiler_params=pltpu.CompilerParams(use_tc_tiling_on_sc=False),
    )
    def agg(src_hbm, dst_hbm, tbl_hbm, zero_hbm, out_hbm, acc,
            sbuf, dbuf, rows, gsem, ssem, *maybe_tbl):
        c = lax.axis_index("c")
        s = lax.axis_index("s")
        bgrp = (c * NS + s) * NGRP
        r0 = s * RPT
        pltpu.sync_copy(zero_hbm.at[pl.ds(r0, RPT)], acc.at[pl.ds(r0, RPT)])
        if spmem_table:
            tbl = maybe_tbl[0]
            pltpu.sync_copy(tbl_hbm.at[pl.ds(r0, RPT)], tbl.at[pl.ds(r0, RPT)])
        else:
            tbl = tbl_hbm
        plsc.subcore_barrier()

        def body(sb, carry):
            g0 = bgrp + sb * SB
            pltpu.sync_copy(src_hbm.at[pl.ds(g0, SB)], sbuf)
            pltpu.sync_copy(dst_hbm.at[pl.ds(g0, SB)], dbuf)
            for j2 in range(SB // nb):
                gd, sd = [], []
                for b in range(nb):
                    j = j2 * nb + b
                    gd.append(
                        pltpu.async_copy(tbl.at[sbuf.at[j, 0]], rows[b], gsem[b])
                    )
                for b in range(nb):
                    j = j2 * nb + b
                    gd[b].wait()
                    sd.append(
                        pltpu.async_copy(rows[b], acc.at[dbuf.at[j, 0]], ssem[b],
                                         add=True)
                    )
                for b in range(nb):
                    sd[b].wait()
            return carry

        lax.fori_loop(0, NSB, body, 0)
        plsc.subcore_barrier()
        pltpu.sync_copy(acc.at[pl.ds(r0, RPT)], out_hbm.at[c].at[pl.ds(r0, RPT)])

    return agg


_agg128 = _make_agg(DF, 3, jnp.bfloat16, True)
_agg32 = _make_agg(DO, 4, jnp.float32, True)


@functools.partial(
    pl.kernel,
    out_type=jax.ShapeDtypeStruct((EPAD,), jnp.float32),
    mesh=_sc_mesh(),
    scratch_types=[
        pltpu.VMEM((SB, 1, G), jnp.int32),
        pltpu.VMEM((SB, 1, G), jnp.int32),
        [pltpu.VMEM((G, DO), jnp.float32) for _ in range(2)],
        [pltpu.VMEM((G, DO), jnp.float32) for _ in range(2)],
        pltpu.VMEM((SB * G,), jnp.float32),
        pltpu.VMEM((16, 16), jnp.float32),
        [pltpu.SemaphoreType.DMA for _ in range(2)],
        [pltpu.SemaphoreType.DMA for _ in range(2)],
        pltpu.VMEM_SHARED((NPAD, DO), jnp.float32),
        pltpu.VMEM_SHARED((NPAD, DO), jnp.float32),
    ],
    compiler_params=pltpu.CompilerParams(
        use_tc_tiling_on_sc=False, needs_layout_passes=False
    ),
)
def _decode_kernel(src_hbm, dst_hbm, zw_hbm, z_hbm, out_hbm,
                   sbuf, dbuf, arows, brows, obuf, tmat, sema, semb, zws, zs):
    c = lax.axis_index("c")
    s = lax.axis_index("s")
    bgrp = (c * NS + s) * NGRP
    r0 = s * RPT
    pltpu.sync_copy(zw_hbm.at[pl.ds(r0, RPT)], zws.at[pl.ds(r0, RPT)])
    pltpu.sync_copy(z_hbm.at[pl.ds(r0, RPT)], zs.at[pl.ds(r0, RPT)])
    plsc.subcore_barrier()
    lanes = lax.broadcasted_iota(jnp.int32, (16,), 0)

    def dot_chunk(j, buf, tmat):
        for e16 in range(G // 16):
            for l in range(16):
                e = e16 * 16 + l
                q = (arows[buf][e, pl.ds(0, 16)] * brows[buf][e, pl.ds(0, 16)]
                     + arows[buf][e, pl.ds(16, 16)] * brows[buf][e, pl.ds(16, 16)])
                tmat[l, pl.ds(0, 16)] = q
            accs = [None] * 4
            for cidx in range(16):
                cvec = jnp.bitwise_and(lanes + cidx, 15)
                p = plsc.load_gather(tmat, [lanes, cvec])
                k = cidx % 4
                accs[k] = p if accs[k] is None else accs[k] + p
            tot = (accs[0] + accs[1]) + (accs[2] + accs[3])
            obuf[pl.ds(j * G + e16 * 16, 16)] = 1.0 / (1.0 + jnp.exp(-tot))

    def wait_pair(b):
        pltpu.make_async_copy(zws.at[sbuf.at[0, 0]], arows[b], sema[b]).wait()
        pltpu.make_async_copy(zs.at[dbuf.at[0, 0]], brows[b], semb[b]).wait()

    def fire_pair(j, b):
        pltpu.async_copy(zws.at[sbuf.at[j, 0]], arows[b], sema[b])
        pltpu.async_copy(zs.at[dbuf.at[j, 0]], brows[b], semb[b])

    def body(sb, carry):
        g0 = bgrp + sb * SB
        pltpu.sync_copy(src_hbm.at[pl.ds(g0, SB)], sbuf)
        pltpu.sync_copy(dst_hbm.at[pl.ds(g0, SB)], dbuf)
        for b in range(2):
            fire_pair(b, b)

        def inner(j2, carry2):
            for b in range(2):
                j = j2 * 2 + b
                wait_pair(b)
                dot_chunk(j, b, tmat)
                fire_pair(j + 2, b)
            return carry2

        lax.fori_loop(0, SB // 2 - 1, inner, 0)
        for b in range(2):
            wait_pair(b)
            dot_chunk(SB - 2 + b, b, tmat)
        pltpu.sync_copy(obuf, out_hbm.at[pl.ds(g0 * G, SB * G)])
        return carry

    lax.fori_loop(0, NSB, body, 0)


_BLK = 256


def _tc1_body(x_ref, wc_ref, da_ref, db_ref, xws_ref, rs_ref):
    deg = jnp.clip(da_ref[...] + db_ref[...], 1.0, None)
    r = lax.rsqrt(deg)
    xw = jnp.dot(x_ref[...], wc_ref[...], preferred_element_type=jnp.float32)
    xws_ref[...] = (xw * r).astype(jnp.bfloat16)
    rs_ref[...] = r


_tc1 = pl.pallas_call(
    _tc1_body,
    grid=(NPAD // _BLK,),
    in_specs=[
        pl.BlockSpec((_BLK, DF), lambda i: (i, 0)),
        pl.BlockSpec((DF, DF), lambda i: (0, 0)),
        pl.BlockSpec((_BLK, 1), lambda i: (i, 0)),
        pl.BlockSpec((_BLK, 1), lambda i: (i, 0)),
    ],
    out_specs=[
        pl.BlockSpec((_BLK, DF), lambda i: (i, 0)),
        pl.BlockSpec((_BLK, 1), lambda i: (i, 0)),
    ],
    out_shape=[
        jax.ShapeDtypeStruct((NPAD, DF), jnp.bfloat16),
        jax.ShapeDtypeStruct((NPAD, 1), jnp.float32),
    ],
)


def _tc2_body(aa_ref, ab_ref, rs_ref, wm_ref, ms_ref):
    r = rs_ref[...]
    agg = aa_ref[...].astype(jnp.float32) + ab_ref[...].astype(jnp.float32)
    h = jnp.maximum(agg * r, 0.0)
    ms_ref[...] = jnp.dot(h, wm_ref[...], preferred_element_type=jnp.float32) * r


_tc2 = pl.pallas_call(
    _tc2_body,
    grid=(NPAD // _BLK,),
    in_specs=[
        pl.BlockSpec((_BLK, DF), lambda i: (i, 0)),
        pl.BlockSpec((_BLK, DF), lambda i: (i, 0)),
        pl.BlockSpec((_BLK, 1), lambda i: (i, 0)),
        pl.BlockSpec((DF, DO), lambda i: (0, 0)),
    ],
    out_specs=pl.BlockSpec((_BLK, DO), lambda i: (i, 0)),
    out_shape=jax.ShapeDtypeStruct((NPAD, DO), jnp.float32),
)


def _tc3_body(za_ref, zb_ref, rs_ref, t_ref, z_ref, zw_ref):
    r = rs_ref[...]
    z = (za_ref[...] + zb_ref[...]) * r
    t = t_ref[0, 0]
    col = lax.broadcasted_iota(jnp.int32, (1, DO), 1)
    w = jnp.where(col < DO // 2, t, 1.0 - t)
    z_ref[...] = z
    zw_ref[...] = z * w


_tc3 = pl.pallas_call(
    _tc3_body,
    grid=(NPAD // _BLK,),
    in_specs=[
        pl.BlockSpec((_BLK, DO), lambda i: (i, 0)),
        pl.BlockSpec((_BLK, DO), lambda i: (i, 0)),
        pl.BlockSpec((_BLK, 1), lambda i: (i, 0)),
        pl.BlockSpec((1, 1), lambda i: (0, 0)),
    ],
    out_specs=[
        pl.BlockSpec((_BLK, DO), lambda i: (i, 0)),
        pl.BlockSpec((_BLK, DO), lambda i: (i, 0)),
    ],
    out_shape=[
        jax.ShapeDtypeStruct((NPAD, DO), jnp.float32),
        jax.ShapeDtypeStruct((NPAD, DO), jnp.float32),
    ],
)


def kernel(x, edge_index, temp, W1, W1mu, W1ls, W2, W2mu, W2ls):
    src = edge_index[0].astype(jnp.int32)
    dst = edge_index[1].astype(jnp.int32)
    padi = jnp.full((EPAD - E,), N, jnp.int32)
    srcf = jnp.concatenate([src, padi])
    dstf = jnp.concatenate([dst, padi])
    srcp = srcf.reshape(EPAD // G, 1, G)
    dstp = dstf.reshape(EPAD // G, 1, G)
    xp = jnp.zeros((NPAD, DF), jnp.float32).at[:N].set(x)
    Wc = jnp.concatenate([W1, W2], axis=1)
    Wm = (jnp.zeros((DF, DO), jnp.float32)
          .at[: DF // 2, : DO // 2].set(W1mu)
          .at[DF // 2 :, DO // 2 :].set(W2mu))
    zero_vec = jnp.zeros((NPAD,), jnp.float32)
    zero128 = jnp.zeros((NPAD, DF), jnp.bfloat16)
    zero32 = jnp.zeros((NPAD, DO), jnp.float32)

    deg2 = _deg_kernel(dstf, zero_vec)
    xws, rs = _tc1(xp, Wc, deg2[0][:, None], deg2[1][:, None])
    agg2 = _agg128(srcp, dstp, xws, zero128)
    ms = _tc2(agg2[0], agg2[1], rs, Wm)
    aggz = _agg32(srcp, dstp, ms, zero32)
    z, zw = _tc3(aggz[0], aggz[1], rs, temp.reshape(1, 1))
    pred = _decode_kernel(srcp, dstp, zw, z)
    return pred[:E]


# speedup vs baseline: 24.5825x; 1.0222x over previous
import functools

import jax
import jax.numpy as jnp
from jax import lax
from jax.experimental import pallas as pl
from jax.experimental.pallas import tpu as pltpu
from jax.experimental.pallas import tpu_sc as plsc

N = 10000
E = 320000
DF = 128
DO = 32
NC, NS = 2, 16
NW = NC * NS
NPAD = 10240
RPT = NPAD // NS
TE = 10240
EPAD = NW * TE
G = 128
NGRP = TE // G
SB = 16
NSB = NGRP // SB


def _sc_mesh():
    return plsc.VectorSubcoreMesh(
        core_axis_name="c", subcore_axis_name="s", num_cores=NC, num_subcores=NS
    )


@functools.partial(
    pl.kernel,
    out_type=jax.ShapeDtypeStruct((NC, NPAD), jnp.float32),
    mesh=_sc_mesh(),
    scratch_types=[
        pltpu.VMEM_SHARED((NPAD,), jnp.float32),
        [pltpu.VMEM((G,), jnp.int32) for _ in range(4)],
        pltpu.VMEM((G,), jnp.float32),
        [pltpu.SemaphoreType.DMA for _ in range(4)],
    ],
)
def _deg_kernel(dst_hbm, zero_hbm, out_hbm, acc, dbuf, ones, sems):
    c = lax.axis_index("c")
    s = lax.axis_index("s")
    base = (c * NS + s) * TE
    r0 = s * RPT
    pltpu.sync_copy(zero_hbm.at[pl.ds(r0, RPT)], acc.at[pl.ds(r0, RPT)])
    for i in range(G // 16):
        ones[pl.ds(i * 16, 16)] = jnp.full((16,), 1.0, jnp.float32)
    plsc.subcore_barrier()

    def body(g4, carry):
        descs = []
        for b in range(4):
            off = base + (g4 * 4 + b) * G
            pltpu.sync_copy(dst_hbm.at[pl.ds(off, G)], dbuf[b])
            descs.append(
                pltpu.async_copy(ones, acc.at[dbuf[b]], sems[b], add=True)
            )
        for d in descs:
            d.wait()
        return carry

    lax.fori_loop(0, NGRP // 4, body, 0)
    plsc.subcore_barrier()
    pltpu.sync_copy(acc.at[pl.ds(r0, RPT)], out_hbm.at[c].at[pl.ds(r0, RPT)])


def _make_agg(width, nb, dtype, spmem_table):
    scratch = [
        pltpu.VMEM_SHARED((NPAD, width), dtype),
        pltpu.VMEM((SB, 1, G), jnp.int32),
        pltpu.VMEM((SB, 1, G), jnp.int32),
        [pltpu.VMEM((G, width), dtype) for _ in range(nb)],
        [pltpu.SemaphoreType.DMA for _ in range(nb)],
        [pltpu.SemaphoreType.DMA for _ in range(nb)],
    ]
    if spmem_table:
        scratch.append(pltpu.VMEM_SHARED((NPAD, width), dtype))

    @functools.partial(
        pl.kernel,
        out_type=jax.ShapeDtypeStruct((NC, NPAD, width), dtype),
        mesh=_sc_mesh(),
        scratch_types=scratch,
        compiler_params=pltpu.CompilerParams(use_tc_tiling_on_sc=False),
    )
    def agg(src_hbm, dst_hbm, tbl_hbm, zero_hbm, out_hbm, acc,
            sbuf, dbuf, rows, gsem, ssem, *maybe_tbl):
        c = lax.axis_index("c")
        s = lax.axis_index("s")
        bgrp = (c * NS + s) * NGRP
        r0 = s * RPT
        pltpu.sync_copy(zero_hbm.at[pl.ds(r0, RPT)], acc.at[pl.ds(r0, RPT)])
        if spmem_table:
            tbl = maybe_tbl[0]
            pltpu.sync_copy(tbl_hbm.at[pl.ds(r0, RPT)], tbl.at[pl.ds(r0, RPT)])
        else:
            tbl = tbl_hbm
        plsc.subcore_barrier()

        def body(sb, carry):
            g0 = bgrp + sb * SB
            pltpu.sync_copy(src_hbm.at[pl.ds(g0, SB)], sbuf)
            pltpu.sync_copy(dst_hbm.at[pl.ds(g0, SB)], dbuf)
            for j2 in range(SB // nb):
                gd, sd = [], []
                for b in range(nb):
                    j = j2 * nb + b
                    gd.append(
                        pltpu.async_copy(tbl.at[sbuf.at[j, 0]], rows[b], gsem[b])
                    )
                for b in range(nb):
                    j = j2 * nb + b
                    gd[b].wait()
                    sd.append(
                        pltpu.async_copy(rows[b], acc.at[dbuf.at[j, 0]], ssem[b],
                                         add=True)
                    )
                for b in range(nb):
                    sd[b].wait()
            return carry

        lax.fori_loop(0, NSB, body, 0)
        plsc.subcore_barrier()
        pltpu.sync_copy(acc.at[pl.ds(r0, RPT)], out_hbm.at[c].at[pl.ds(r0, RPT)])

    return agg


_agg128 = _make_agg(DF, 3, jnp.bfloat16, True)
_agg32 = _make_agg(DO, 4, jnp.float32, True)


@functools.partial(
    pl.kernel,
    out_type=jax.ShapeDtypeStruct((EPAD,), jnp.float32),
    mesh=_sc_mesh(),
    scratch_types=[
        pltpu.VMEM((SB, 1, G), jnp.int32),
        pltpu.VMEM((SB, 1, G), jnp.int32),
        [pltpu.VMEM((G, DO), jnp.float32) for _ in range(2)],
        [pltpu.VMEM((G, DO), jnp.float32) for _ in range(2)],
        pltpu.VMEM((SB * G,), jnp.float32),
        pltpu.VMEM((16, 16), jnp.float32),
        [pltpu.SemaphoreType.DMA for _ in range(2)],
        [pltpu.SemaphoreType.DMA for _ in range(2)],
        pltpu.VMEM_SHARED((NPAD, DO), jnp.float32),
        pltpu.VMEM_SHARED((NPAD, DO), jnp.float32),
        pltpu.VMEM((G, DO), jnp.float32),
        pltpu.VMEM((G, DO), jnp.float32),
        pltpu.VMEM((G, 1), jnp.float32),
        pltpu.VMEM((G, DO), jnp.float32),
        pltpu.VMEM((G, DO), jnp.float32),
        pltpu.VMEM((16,), jnp.float32),
    ],
    compiler_params=pltpu.CompilerParams(
        use_tc_tiling_on_sc=False, needs_layout_passes=False
    ),
)
def _decode_kernel(src_hbm, dst_hbm, aza_hbm, azb_hbm, rs_hbm, tvec_hbm, out_hbm,
                   sbuf, dbuf, arows, brows, obuf, tmat, sema, semb, zws, zs,
                   za_c, zb_c, rs_c, zbuf, zwbuf, tv):
    c = lax.axis_index("c")
    s = lax.axis_index("s")
    bgrp = (c * NS + s) * NGRP
    r0 = s * RPT
    pltpu.sync_copy(tvec_hbm, tv)
    for ci in range(RPT // G):
        rr = r0 + ci * G
        pltpu.sync_copy(aza_hbm.at[pl.ds(rr, G)], za_c)
        pltpu.sync_copy(azb_hbm.at[pl.ds(rr, G)], zb_c)
        pltpu.sync_copy(rs_hbm.at[pl.ds(rr, G)], rs_c)
        tvv = tv[pl.ds(0, 16)]
        zeros16 = jnp.zeros((16,), jnp.int32)

        def stage_row(r, carry):
            rsc = plsc.load_gather(rs_c, [jnp.full((16,), r, jnp.int32), zeros16])
            zlo = (za_c[r, pl.ds(0, 16)] + zb_c[r, pl.ds(0, 16)]) * rsc
            zhi = (za_c[r, pl.ds(16, 16)] + zb_c[r, pl.ds(16, 16)]) * rsc
            zbuf[r, pl.ds(0, 16)] = zlo
            zbuf[r, pl.ds(16, 16)] = zhi
            zwbuf[r, pl.ds(0, 16)] = zlo * tvv
            zwbuf[r, pl.ds(16, 16)] = zhi * (1.0 - tvv)
            return carry

        lax.fori_loop(0, G, stage_row, 0)
        pltpu.sync_copy(zbuf, zs.at[pl.ds(rr, G)])
        pltpu.sync_copy(zwbuf, zws.at[pl.ds(rr, G)])
    plsc.subcore_barrier()
    lanes = lax.broadcasted_iota(jnp.int32, (16,), 0)

    def dot_chunk(j, buf, tmat):
        for e16 in range(G // 16):
            for l in range(16):
                e = e16 * 16 + l
                q = (arows[buf][e, pl.ds(0, 16)] * brows[buf][e, pl.ds(0, 16)]
                     + arows[buf][e, pl.ds(16, 16)] * brows[buf][e, pl.ds(16, 16)])
                tmat[l, pl.ds(0, 16)] = q
            accs = [None] * 4
            for cidx in range(16):
                cvec = jnp.bitwise_and(lanes + cidx, 15)
                p = plsc.load_gather(tmat, [lanes, cvec])
                k = cidx % 4
                accs[k] = p if accs[k] is None else accs[k] + p
            tot = (accs[0] + accs[1]) + (accs[2] + accs[3])
            obuf[pl.ds(j * G + e16 * 16, 16)] = 1.0 / (1.0 + jnp.exp(-tot))

    def wait_pair(b):
        pltpu.make_async_copy(zws.at[sbuf.at[0, 0]], arows[b], sema[b]).wait()
        pltpu.make_async_copy(zs.at[dbuf.at[0, 0]], brows[b], semb[b]).wait()

    def fire_pair(j, b):
        pltpu.async_copy(zws.at[sbuf.at[j, 0]], arows[b], sema[b])
        pltpu.async_copy(zs.at[dbuf.at[j, 0]], brows[b], semb[b])

    def body(sb, carry):
        g0 = bgrp + sb * SB
        pltpu.sync_copy(src_hbm.at[pl.ds(g0, SB)], sbuf)
        pltpu.sync_copy(dst_hbm.at[pl.ds(g0, SB)], dbuf)
        for b in range(2):
            fire_pair(b, b)

        def inner(j2, carry2):
            for b in range(2):
                j = j2 * 2 + b
                wait_pair(b)
                dot_chunk(j, b, tmat)
                fire_pair(j + 2, b)
            return carry2

        lax.fori_loop(0, SB // 2 - 1, inner, 0)
        for b in range(2):
            wait_pair(b)
            dot_chunk(SB - 2 + b, b, tmat)
        pltpu.sync_copy(obuf, out_hbm.at[pl.ds(g0 * G, SB * G)])
        return carry

    lax.fori_loop(0, NSB, body, 0)


_BLK = 256


def _tc1_body(x_ref, wc_ref, da_ref, db_ref, xws_ref, rs_ref):
    deg = jnp.clip(da_ref[...] + db_ref[...], 1.0, None)
    r = lax.rsqrt(deg)
    xw = jnp.dot(x_ref[...], wc_ref[...], preferred_element_type=jnp.float32)
    xws_ref[...] = (xw * r).astype(jnp.bfloat16)
    rs_ref[...] = r


_tc1 = pl.pallas_call(
    _tc1_body,
    grid=(NPAD // _BLK,),
    in_specs=[
        pl.BlockSpec((_BLK, DF), lambda i: (i, 0)),
        pl.BlockSpec((DF, DF), lambda i: (0, 0)),
        pl.BlockSpec((_BLK, 1), lambda i: (i, 0)),
        pl.BlockSpec((_BLK, 1), lambda i: (i, 0)),
    ],
    out_specs=[
        pl.BlockSpec((_BLK, DF), lambda i: (i, 0)),
        pl.BlockSpec((_BLK, 1), lambda i: (i, 0)),
    ],
    out_shape=[
        jax.ShapeDtypeStruct((NPAD, DF), jnp.bfloat16),
        jax.ShapeDtypeStruct((NPAD, 1), jnp.float32),
    ],
)


def _tc2_body(aa_ref, ab_ref, rs_ref, wm_ref, ms_ref):
    r = rs_ref[...]
    agg = aa_ref[...].astype(jnp.float32) + ab_ref[...].astype(jnp.float32)
    h = jnp.maximum(agg * r, 0.0)
    ms_ref[...] = jnp.dot(h, wm_ref[...], preferred_element_type=jnp.float32) * r


_tc2 = pl.pallas_call(
    _tc2_body,
    grid=(NPAD // _BLK,),
    in_specs=[
        pl.BlockSpec((_BLK, DF), lambda i: (i, 0)),
        pl.BlockSpec((_BLK, DF), lambda i: (i, 0)),
        pl.BlockSpec((_BLK, 1), lambda i: (i, 0)),
        pl.BlockSpec((DF, DO), lambda i: (0, 0)),
    ],
    out_specs=pl.BlockSpec((_BLK, DO), lambda i: (i, 0)),
    out_shape=jax.ShapeDtypeStruct((NPAD, DO), jnp.float32),
)


def kernel(x, edge_index, temp, W1, W1mu, W1ls, W2, W2mu, W2ls):
    src = edge_index[0].astype(jnp.int32)
    dst = edge_index[1].astype(jnp.int32)
    padi = jnp.full((EPAD - E,), N, jnp.int32)
    srcf = jnp.concatenate([src, padi])
    dstf = jnp.concatenate([dst, padi])
    srcp = srcf.reshape(EPAD // G, 1, G)
    dstp = dstf.reshape(EPAD // G, 1, G)
    xp = jnp.zeros((NPAD, DF), jnp.float32).at[:N].set(x)
    Wc = jnp.concatenate([W1, W2], axis=1)
    Wm = (jnp.zeros((DF, DO), jnp.float32)
          .at[: DF // 2, : DO // 2].set(W1mu)
          .at[DF // 2 :, DO // 2 :].set(W2mu))
    zero_vec = jnp.zeros((NPAD,), jnp.float32)
    zero128 = jnp.zeros((NPAD, DF), jnp.bfloat16)
    zero32 = jnp.zeros((NPAD, DO), jnp.float32)

    deg2 = _deg_kernel(dstf, zero_vec)
    xws, rs = _tc1(xp, Wc, deg2[0][:, None], deg2[1][:, None])
    agg2 = _agg128(srcp, dstp, xws, zero128)
    ms = _tc2(agg2[0], agg2[1], rs, Wm)
    aggz = _agg32(srcp, dstp, ms, zero32)
    tvec = jnp.broadcast_to(temp, (16,))
    pred = _decode_kernel(srcp, dstp, aggz[0], aggz[1], rs, tvec)
    return pred[:E]


# speedup vs baseline: 24.7038x; 1.0049x over previous
import functools

import jax
import jax.numpy as jnp
from jax import lax
from jax.experimental import pallas as pl
from jax.experimental.pallas import tpu as pltpu
from jax.experimental.pallas import tpu_sc as plsc

N = 10000
E = 320000
DF = 128
DO = 32
NC, NS = 2, 16
NW = NC * NS
NPAD = 10240
RPT = NPAD // NS
TE = 10240
EPAD = NW * TE
G = 128
NGRP = TE // G
SB = 16
NSB = NGRP // SB


def _sc_mesh():
    return plsc.VectorSubcoreMesh(
        core_axis_name="c", subcore_axis_name="s", num_cores=NC, num_subcores=NS
    )


@functools.partial(
    pl.kernel,
    out_type=jax.ShapeDtypeStruct((NC, NPAD), jnp.float32),
    mesh=_sc_mesh(),
    scratch_types=[
        pltpu.VMEM_SHARED((NPAD,), jnp.float32),
        [pltpu.VMEM((G,), jnp.int32) for _ in range(4)],
        pltpu.VMEM((G,), jnp.float32),
        [pltpu.SemaphoreType.DMA for _ in range(4)],
    ],
)
def _deg_kernel(dst_hbm, zero_hbm, out_hbm, acc, dbuf, ones, sems):
    c = lax.axis_index("c")
    s = lax.axis_index("s")
    base = (c * NS + s) * TE
    r0 = s * RPT
    pltpu.sync_copy(zero_hbm.at[pl.ds(r0, RPT)], acc.at[pl.ds(r0, RPT)])
    for i in range(G // 16):
        ones[pl.ds(i * 16, 16)] = jnp.full((16,), 1.0, jnp.float32)
    plsc.subcore_barrier()

    def body(g4, carry):
        descs = []
        for b in range(4):
            off = base + (g4 * 4 + b) * G
            pltpu.sync_copy(dst_hbm.at[pl.ds(off, G)], dbuf[b])
            descs.append(
                pltpu.async_copy(ones, acc.at[dbuf[b]], sems[b], add=True)
            )
        for d in descs:
            d.wait()
        return carry

    lax.fori_loop(0, NGRP // 4, body, 0)
    plsc.subcore_barrier()
    pltpu.sync_copy(acc.at[pl.ds(r0, RPT)], out_hbm.at[c].at[pl.ds(r0, RPT)])


def _make_agg(width, nb, dtype, spmem_table):
    scratch = [
        pltpu.VMEM_SHARED((NPAD, width), dtype),
        pltpu.VMEM((SB, 1, G), jnp.int32),
        pltpu.VMEM((SB, 1, G), jnp.int32),
        [pltpu.VMEM((G, width), dtype) for _ in range(nb)],
        [pltpu.SemaphoreType.DMA for _ in range(nb)],
        [pltpu.SemaphoreType.DMA for _ in range(nb)],
    ]
    if spmem_table:
        scratch.append(pltpu.VMEM_SHARED((NPAD, width), dtype))

    @functools.partial(
        pl.kernel,
        out_type=jax.ShapeDtypeStruct((NC, NPAD, width), dtype),
        mesh=_sc_mesh(),
        scratch_types=scratch,
        compiler_params=pltpu.CompilerParams(use_tc_tiling_on_sc=False),
    )
    def agg(src_hbm, dst_hbm, tbl_hbm, zero_hbm, out_hbm, acc,
            sbuf, dbuf, rows, gsem, ssem, *maybe_tbl):
        c = lax.axis_index("c")
        s = lax.axis_index("s")
        bgrp = (c * NS + s) * NGRP
        r0 = s * RPT
        pltpu.sync_copy(zero_hbm.at[pl.ds(r0, RPT)], acc.at[pl.ds(r0, RPT)])
        if spmem_table:
            tbl = maybe_tbl[0]
            pltpu.sync_copy(tbl_hbm.at[pl.ds(r0, RPT)], tbl.at[pl.ds(r0, RPT)])
        else:
            tbl = tbl_hbm
        plsc.subcore_barrier()

        def body(sb, carry):
            g0 = bgrp + sb * SB
            pltpu.sync_copy(src_hbm.at[pl.ds(g0, SB)], sbuf)
            pltpu.sync_copy(dst_hbm.at[pl.ds(g0, SB)], dbuf)
            for j2 in range(SB // nb):
                gd, sd = [], []
                for b in range(nb):
                    j = j2 * nb + b
                    gd.append(
                        pltpu.async_copy(tbl.at[sbuf.at[j, 0]], rows[b], gsem[b])
                    )
                for b in range(nb):
                    j = j2 * nb + b
                    gd[b].wait()
                    sd.append(
                        pltpu.async_copy(rows[b], acc.at[dbuf.at[j, 0]], ssem[b],
                                         add=True)
                    )
                for b in range(nb):
                    sd[b].wait()
            return carry

        lax.fori_loop(0, NSB, body, 0)
        plsc.subcore_barrier()
        pltpu.sync_copy(acc.at[pl.ds(r0, RPT)], out_hbm.at[c].at[pl.ds(r0, RPT)])

    return agg


_agg128 = _make_agg(DF, 3, jnp.bfloat16, True)
_agg32 = _make_agg(DO, 4, jnp.float32, True)


@functools.partial(
    pl.kernel,
    out_type=jax.ShapeDtypeStruct((EPAD,), jnp.float32),
    mesh=_sc_mesh(),
    scratch_types=[
        pltpu.VMEM((SB, 1, G), jnp.int32),
        pltpu.VMEM((SB, 1, G), jnp.int32),
        [pltpu.VMEM((G, DO), jnp.float32) for _ in range(2)],
        [pltpu.VMEM((G, DO), jnp.float32) for _ in range(2)],
        pltpu.VMEM((SB * G,), jnp.float32),
        [pltpu.VMEM((16, 16), jnp.float32) for _ in range(2)],
        [pltpu.SemaphoreType.DMA for _ in range(2)],
        [pltpu.SemaphoreType.DMA for _ in range(2)],
        pltpu.VMEM_SHARED((NPAD, DO), jnp.float32),
        pltpu.VMEM_SHARED((NPAD, DO), jnp.float32),
        pltpu.VMEM((G, DO), jnp.float32),
        pltpu.VMEM((G, DO), jnp.float32),
        pltpu.VMEM((G, 1), jnp.float32),
        pltpu.VMEM((G, DO), jnp.float32),
        pltpu.VMEM((G, DO), jnp.float32),
        pltpu.VMEM((16,), jnp.float32),
    ],
    compiler_params=pltpu.CompilerParams(
        use_tc_tiling_on_sc=False, needs_layout_passes=False
    ),
)
def _decode_kernel(src_hbm, dst_hbm, aza_hbm, azb_hbm, rs_hbm, tvec_hbm, out_hbm,
                   sbuf, dbuf, arows, brows, obuf, tmat, sema, semb, zws, zs,
                   za_c, zb_c, rs_c, zbuf, zwbuf, tv):
    c = lax.axis_index("c")
    s = lax.axis_index("s")
    bgrp = (c * NS + s) * NGRP
    r0 = s * RPT
    pltpu.sync_copy(tvec_hbm, tv)
    for ci in range(RPT // G):
        rr = r0 + ci * G
        pltpu.sync_copy(aza_hbm.at[pl.ds(rr, G)], za_c)
        pltpu.sync_copy(azb_hbm.at[pl.ds(rr, G)], zb_c)
        pltpu.sync_copy(rs_hbm.at[pl.ds(rr, G)], rs_c)
        tvv = tv[pl.ds(0, 16)]
        zeros16 = jnp.zeros((16,), jnp.int32)

        def stage_row(r, carry):
            rsc = plsc.load_gather(rs_c, [jnp.full((16,), r, jnp.int32), zeros16])
            zlo = (za_c[r, pl.ds(0, 16)] + zb_c[r, pl.ds(0, 16)]) * rsc
            zhi = (za_c[r, pl.ds(16, 16)] + zb_c[r, pl.ds(16, 16)]) * rsc
            zbuf[r, pl.ds(0, 16)] = zlo
            zbuf[r, pl.ds(16, 16)] = zhi
            zwbuf[r, pl.ds(0, 16)] = zlo * tvv
            zwbuf[r, pl.ds(16, 16)] = zhi * (1.0 - tvv)
            return carry

        lax.fori_loop(0, G, stage_row, 0)
        pltpu.sync_copy(zbuf, zs.at[pl.ds(rr, G)])
        pltpu.sync_copy(zwbuf, zws.at[pl.ds(rr, G)])
    plsc.subcore_barrier()
    lanes = lax.broadcasted_iota(jnp.int32, (16,), 0)

    def dot_chunk(j, buf, tmat):
        for e16 in range(G // 16):
            tm = tmat[e16 % 2]
            for l in range(16):
                e = e16 * 16 + l
                q = (arows[buf][e, pl.ds(0, 16)] * brows[buf][e, pl.ds(0, 16)]
                     + arows[buf][e, pl.ds(16, 16)] * brows[buf][e, pl.ds(16, 16)])
                tm[l, pl.ds(0, 16)] = q
            accs = [None] * 4
            for cidx in range(16):
                cvec = jnp.bitwise_and(lanes + cidx, 15)
                p = plsc.load_gather(tm, [lanes, cvec])
                k = cidx % 4
                accs[k] = p if accs[k] is None else accs[k] + p
            tot = (accs[0] + accs[1]) + (accs[2] + accs[3])
            obuf[pl.ds(j * G + e16 * 16, 16)] = 1.0 / (1.0 + jnp.exp(-tot))

    def wait_pair(b):
        pltpu.make_async_copy(zws.at[sbuf.at[0, 0]], arows[b], sema[b]).wait()
        pltpu.make_async_copy(zs.at[dbuf.at[0, 0]], brows[b], semb[b]).wait()

    def fire_pair(j, b):
        pltpu.async_copy(zws.at[sbuf.at[j, 0]], arows[b], sema[b])
        pltpu.async_copy(zs.at[dbuf.at[j, 0]], brows[b], semb[b])

    def body(sb, carry):
        g0 = bgrp + sb * SB
        pltpu.sync_copy(src_hbm.at[pl.ds(g0, SB)], sbuf)
        pltpu.sync_copy(dst_hbm.at[pl.ds(g0, SB)], dbuf)
        for b in range(2):
            fire_pair(b, b)

        def inner(j2, carry2):
            for b in range(2):
                j = j2 * 2 + b
                wait_pair(b)
                dot_chunk(j, b, tmat)
                fire_pair(j + 2, b)
            return carry2

        lax.fori_loop(0, SB // 2 - 1, inner, 0)
        for b in range(2):
            wait_pair(b)
            dot_chunk(SB - 2 + b, b, tmat)
        pltpu.sync_copy(obuf, out_hbm.at[pl.ds(g0 * G, SB * G)])
        return carry

    lax.fori_loop(0, NSB, body, 0)


_BLK = 256


def _tc1_body(x_ref, wc_ref, da_ref, db_ref, xws_ref, rs_ref):
    deg = jnp.clip(da_ref[...] + db_ref[...], 1.0, None)
    r = lax.rsqrt(deg)
    xw = jnp.dot(x_ref[...], wc_ref[...], preferred_element_type=jnp.float32)
    xws_ref[...] = (xw * r).astype(jnp.bfloat16)
    rs_ref[...] = r


_tc1 = pl.pallas_call(
    _tc1_body,
    grid=(NPAD // _BLK,),
    in_specs=[
        pl.BlockSpec((_BLK, DF), lambda i: (i, 0)),
        pl.BlockSpec((DF, DF), lambda i: (0, 0)),
        pl.BlockSpec((_BLK, 1), lambda i: (i, 0)),
        pl.BlockSpec((_BLK, 1), lambda i: (i, 0)),
    ],
    out_specs=[
        pl.BlockSpec((_BLK, DF), lambda i: (i, 0)),
        pl.BlockSpec((_BLK, 1), lambda i: (i, 0)),
    ],
    out_shape=[
        jax.ShapeDtypeStruct((NPAD, DF), jnp.bfloat16),
        jax.ShapeDtypeStruct((NPAD, 1), jnp.float32),
    ],
)


def _tc2_body(aa_ref, ab_ref, rs_ref, wm_ref, ms_ref):
    r = rs_ref[...]
    agg = aa_ref[...].astype(jnp.float32) + ab_ref[...].astype(jnp.float32)
    h = jnp.maximum(agg * r, 0.0)
    ms_ref[...] = jnp.dot(h, wm_ref[...], preferred_element_type=jnp.float32) * r


_tc2 = pl.pallas_call(
    _tc2_body,
    grid=(NPAD // _BLK,),
    in_specs=[
        pl.BlockSpec((_BLK, DF), lambda i: (i, 0)),
        pl.BlockSpec((_BLK, DF), lambda i: (i, 0)),
        pl.BlockSpec((_BLK, 1), lambda i: (i, 0)),
        pl.BlockSpec((DF, DO), lambda i: (0, 0)),
    ],
    out_specs=pl.BlockSpec((_BLK, DO), lambda i: (i, 0)),
    out_shape=jax.ShapeDtypeStruct((NPAD, DO), jnp.float32),
)


def kernel(x, edge_index, temp, W1, W1mu, W1ls, W2, W2mu, W2ls):
    src = edge_index[0].astype(jnp.int32)
    dst = edge_index[1].astype(jnp.int32)
    padi = jnp.full((EPAD - E,), N, jnp.int32)
    srcf = jnp.concatenate([src, padi])
    dstf = jnp.concatenate([dst, padi])
    srcp = srcf.reshape(EPAD // G, 1, G)
    dstp = dstf.reshape(EPAD // G, 1, G)
    xp = jnp.zeros((NPAD, DF), jnp.float32).at[:N].set(x)
    Wc = jnp.concatenate([W1, W2], axis=1)
    Wm = (jnp.zeros((DF, DO), jnp.float32)
          .at[: DF // 2, : DO // 2].set(W1mu)
          .at[DF // 2 :, DO // 2 :].set(W2mu))
    zero_vec = jnp.zeros((NPAD,), jnp.float32)
    zero128 = jnp.zeros((NPAD, DF), jnp.bfloat16)
    zero32 = jnp.zeros((NPAD, DO), jnp.float32)

    deg2 = _deg_kernel(dstf, zero_vec)
    xws, rs = _tc1(xp, Wc, deg2[0][:, None], deg2[1][:, None])
    agg2 = _agg128(srcp, dstp, xws, zero128)
    ms = _tc2(agg2[0], agg2[1], rs, Wm)
    aggz = _agg32(srcp, dstp, ms, zero32)
    tvec = jnp.broadcast_to(temp, (16,))
    pred = _decode_kernel(srcp, dstp, aggz[0], aggz[1], rs, tvec)
    return pred[:E]
